# Initial kernel scaffold; baseline (speedup 1.0000x reference)
import jax
import jax.numpy as jnp
from jax.experimental import pallas as pl


def kernel(user_reviews, item_reviews, uids, iids, user_item2id, item_user2id, user_item_ratio, item_user_ratio, user_doc, item_doc, type, month, user_word_table, item_word_table, user_id_table, item_id_table, type_table, month_table, Wu, bu, Wi, bi):
    raise NotImplementedError("write your pallas kernel here")


# speedup vs baseline: 16.8700x; 16.8700x over previous
import functools

import jax
import jax.numpy as jnp
from jax import lax
from jax.experimental import pallas as pl
from jax.experimental.pallas import tpu as pltpu
from jax.experimental.pallas import tpu_sc as plsc

_B = 1024
_UMR = 20
_IMR = 20
_RML = 50
_D = 64
_NC = 2
_NS = 16
_NW = _NC * _NS
_SEG = _B * _UMR
_SEG_W = _SEG // _NW
_IDX_W = _SEG_W * _RML
_GB = 4
_ROWS = _GB * _RML
_NBLK = _SEG_W // _GB
_IDB = _B // _NW
_F32 = jnp.float32


def _build_sc_pool():
    mesh = plsc.VectorSubcoreMesh(core_axis_name="c", subcore_axis_name="s")
    out_type = (
        jax.ShapeDtypeStruct((_SEG, _D), _F32),
        jax.ShapeDtypeStruct((_SEG, _D), _F32),
        jax.ShapeDtypeStruct((_B, _D), _F32),
        jax.ShapeDtypeStruct((_B, _D), _F32),
        jax.ShapeDtypeStruct((_B, _D), _F32),
        jax.ShapeDtypeStruct((_B, _D), _F32),
    )
    scratch_types = [
        pltpu.VMEM((_IDX_W,), jnp.int32),
        pltpu.VMEM((2, _ROWS, _D), _F32),
        pltpu.VMEM((_SEG_W, _D), _F32),
        pltpu.VMEM((_IDB,), jnp.int32),
        pltpu.VMEM((_IDB, _D), _F32),
        pltpu.SemaphoreType.DMA,
        pltpu.SemaphoreType.DMA,
    ]

    @functools.partial(
        pl.kernel, out_type=out_type, mesh=mesh, scratch_types=scratch_types,
        compiler_params=pltpu.CompilerParams(use_tc_tiling_on_sc=False))
    def sc_pool(ur_idx, ir_idx, uwt, iwt, uids, iids, tps, mns,
                uid_t, iid_t, t_t, m_t,
                pu_out, pi_out, ue_out, ie_out, te_out, me_out,
                idx_v, rows_v, pool_v, sid_v, srow_v, sem_a, sem_b):
        wid = lax.axis_index("s") * _NC + lax.axis_index("c")
        sems = (sem_a, sem_b)

        def run_side(idx_hbm, tab_hbm, out_hbm):
            pltpu.sync_copy(idx_hbm.at[pl.ds(wid * _IDX_W, _IDX_W)], idx_v)

            def start(blk, b):
                off = pl.multiple_of(blk * _ROWS, 8)
                pltpu.make_async_copy(
                    tab_hbm.at[idx_v.at[pl.ds(off, _ROWS)]],
                    rows_v.at[b], sems[b]).start()

            def wait(b):
                pltpu.make_async_copy(
                    tab_hbm.at[idx_v.at[pl.ds(0, _ROWS)]],
                    rows_v.at[b], sems[b]).wait()

            start(0, 0)
            start(1, 1)

            def block_body(t, carry):
                for b in range(2):
                    blk = 2 * t + b
                    wait(b)
                    rows = rows_v.at[b]
                    for s in range(_GB):
                        base = s * _RML
                        acc = tuple(rows[base, pl.ds(c * 16, 16)]
                                    for c in range(4))

                        def row_body(r, a, base=base, rows=rows):
                            return tuple(
                                a[c] + rows[base + r, pl.ds(c * 16, 16)]
                                for c in range(4))

                        acc = lax.fori_loop(1, _RML, row_body, acc)
                        seg = blk * _GB + s
                        for c in range(4):
                            pool_v[seg, pl.ds(c * 16, 16)] = acc[c]
                    nblk = blk + 2

                    @pl.when(nblk < _NBLK)
                    def _():
                        start(nblk, b)
                return carry

            lax.fori_loop(0, _NBLK // 2, block_body, 0)
            pltpu.sync_copy(pool_v, out_hbm.at[pl.ds(wid * _SEG_W, _SEG_W)])

        def id_gather(idx_hbm, tab_hbm, out_hbm):
            pltpu.sync_copy(idx_hbm.at[pl.ds(wid * _IDB, _IDB)], sid_v)
            pltpu.sync_copy(tab_hbm.at[sid_v], srow_v)
            pltpu.sync_copy(srow_v, out_hbm.at[pl.ds(wid * _IDB, _IDB)])

        run_side(ur_idx, uwt, pu_out)
        run_side(ir_idx, iwt, pi_out)
        id_gather(uids, uid_t, ue_out)
        id_gather(iids, iid_t, ie_out)
        id_gather(tps, t_t, te_out)
        id_gather(mns, m_t, me_out)

    return sc_pool


_sc_pool = _build_sc_pool()

_BB = 256


def _dense_body(pu_ref, pi_ref, wu_ref, wi_ref, bu_ref, bi_ref,
                ue_ref, ie_ref, te_ref, me_ref, uf_ref, if_ref):
    def side(x_ref, w_ref, b_ref, id_ref, extra_ref, out_ref, nrev):
        w = w_ref[...]
        bvec = b_ref[...]
        acc = jnp.zeros((_BB, _D), _F32)
        for r in range(nrev):
            x = x_ref[:, r, :]
            h = jnp.dot(x, w, preferred_element_type=_F32) + bvec
            acc = acc + jnp.maximum(h, 0.0)
        out_ref[:, 0, :] = id_ref[...]
        out_ref[:, 1, :] = extra_ref[...]
        out_ref[:, 2, :] = acc

    side(pu_ref, wu_ref, bu_ref, ue_ref, te_ref, uf_ref, _UMR)
    side(pi_ref, wi_ref, bi_ref, ie_ref, me_ref, if_ref, _IMR)


def _dense(pu3, pi3, Wu, Wi, bu2, bi2, ue, ie, te, me):
    grid = (_B // _BB,)
    return pl.pallas_call(
        _dense_body,
        grid=grid,
        in_specs=[
            pl.BlockSpec((_BB, _UMR, _D), lambda i: (i, 0, 0)),
            pl.BlockSpec((_BB, _IMR, _D), lambda i: (i, 0, 0)),
            pl.BlockSpec((_D, _D), lambda i: (0, 0)),
            pl.BlockSpec((_D, _D), lambda i: (0, 0)),
            pl.BlockSpec((1, _D), lambda i: (0, 0)),
            pl.BlockSpec((1, _D), lambda i: (0, 0)),
            pl.BlockSpec((_BB, _D), lambda i: (i, 0)),
            pl.BlockSpec((_BB, _D), lambda i: (i, 0)),
            pl.BlockSpec((_BB, _D), lambda i: (i, 0)),
            pl.BlockSpec((_BB, _D), lambda i: (i, 0)),
        ],
        out_specs=[
            pl.BlockSpec((_BB, 3, _D), lambda i: (i, 0, 0)),
            pl.BlockSpec((_BB, 3, _D), lambda i: (i, 0, 0)),
        ],
        out_shape=[
            jax.ShapeDtypeStruct((_B, 3, _D), _F32),
            jax.ShapeDtypeStruct((_B, 3, _D), _F32),
        ],
    )(pu3, pi3, Wu, Wi, bu2, bi2, ue, ie, te, me)


def kernel(user_reviews, item_reviews, uids, iids, user_item2id, item_user2id,
           user_item_ratio, item_user_ratio, user_doc, item_doc, type, month,
           user_word_table, item_word_table, user_id_table, item_id_table,
           type_table, month_table, Wu, bu, Wi, bi):
    ur_idx = user_reviews.reshape(-1).astype(jnp.int32)
    ir_idx = item_reviews.reshape(-1).astype(jnp.int32)
    pu, pi, ue, ie, te, me = _sc_pool(
        ur_idx, ir_idx, user_word_table, item_word_table,
        uids.astype(jnp.int32), iids.astype(jnp.int32),
        type.astype(jnp.int32), month.astype(jnp.int32),
        user_id_table, item_id_table, type_table, month_table)
    user_fea, item_fea = _dense(
        pu.reshape(_B, _UMR, _D), pi.reshape(_B, _IMR, _D),
        Wu, Wi, bu.reshape(1, _D), bi.reshape(1, _D), ue, ie, te, me)
    return (user_fea, item_fea)


# speedup vs baseline: 19.4033x; 1.1502x over previous
import functools

import jax
import jax.numpy as jnp
from jax import lax
from jax.experimental import pallas as pl
from jax.experimental.pallas import tpu as pltpu
from jax.experimental.pallas import tpu_sc as plsc

_B = 1024
_UMR = 20
_IMR = 20
_RML = 50
_D = 64
_TYPE_N = 10
_MONTH_N = 12
_NC = 2
_NS = 16
_NW = _NC * _NS
_SEG = _B * _UMR
_SEG_W = _SEG // _NW
_IDX_W = _SEG_W * _RML
_GB = 4
_ROWS = _GB * _RML
_NBLK = _SEG_W // _GB
_F32 = jnp.float32


def _build_sc_pool():
    mesh = plsc.VectorSubcoreMesh(core_axis_name="c", subcore_axis_name="s")
    out_type = (
        jax.ShapeDtypeStruct((_SEG, _D), _F32),
        jax.ShapeDtypeStruct((_SEG, _D), _F32),
    )
    scratch_types = [
        pltpu.VMEM((_IDX_W,), jnp.int32),
        pltpu.VMEM((2, _ROWS, _D), _F32),
        pltpu.VMEM((_SEG_W, _D), _F32),
        pltpu.SemaphoreType.DMA,
        pltpu.SemaphoreType.DMA,
    ]

    @functools.partial(
        pl.kernel, out_type=out_type, mesh=mesh, scratch_types=scratch_types,
        compiler_params=pltpu.CompilerParams(use_tc_tiling_on_sc=False))
    def sc_pool(ur_idx, ir_idx, uwt, iwt,
                pu_out, pi_out,
                idx_v, rows_v, pool_v, sem_a, sem_b):
        wid = lax.axis_index("s") * _NC + lax.axis_index("c")
        sems = (sem_a, sem_b)

        def run_side(idx_hbm, tab_hbm, out_hbm):
            pltpu.sync_copy(idx_hbm.at[pl.ds(wid * _IDX_W, _IDX_W)], idx_v)

            def start(blk, b):
                off = pl.multiple_of(blk * _ROWS, 8)
                pltpu.make_async_copy(
                    tab_hbm.at[idx_v.at[pl.ds(off, _ROWS)]],
                    rows_v.at[b], sems[b]).start()

            def wait(b):
                pltpu.make_async_copy(
                    tab_hbm.at[idx_v.at[pl.ds(0, _ROWS)]],
                    rows_v.at[b], sems[b]).wait()

            start(0, 0)
            start(1, 1)

            def block_body(t, carry):
                for b in range(2):
                    blk = 2 * t + b
                    wait(b)
                    rows = rows_v.at[b]
                    for s in range(_GB):
                        base = s * _RML
                        acc = tuple(rows[base, pl.ds(c * 16, 16)]
                                    for c in range(4))

                        def row_body(r, a, base=base, rows=rows):
                            return tuple(
                                a[c] + rows[base + r, pl.ds(c * 16, 16)]
                                for c in range(4))

                        acc = lax.fori_loop(1, _RML, row_body, acc)
                        seg = blk * _GB + s
                        for c in range(4):
                            pool_v[seg, pl.ds(c * 16, 16)] = acc[c]
                    nblk = blk + 2

                    @pl.when(nblk < _NBLK)
                    def _():
                        start(nblk, b)
                return carry

            lax.fori_loop(0, _NBLK // 2, block_body, 0)
            pltpu.sync_copy(pool_v, out_hbm.at[pl.ds(wid * _SEG_W, _SEG_W)])

        run_side(ur_idx, uwt, pu_out)
        run_side(ir_idx, iwt, pi_out)

    return sc_pool


_sc_pool = _build_sc_pool()

_BB = 256


def _dense_body(uids_sm, iids_sm,
                pu_ref, pi_ref, wu_ref, wi_ref, bu_ref, bi_ref,
                uidtab_ref, iidtab_ref, ttab_ref, mtab_ref,
                typ_ref, mon_ref, uf_ref, if_ref,
                urow_v, irow_v, sem_u, sem_i):
    i = pl.program_id(0)

    def id_copy(ids_sm, tab_ref, row_v, sem, j):
        rid = ids_sm[i * _BB + j]
        return pltpu.make_async_copy(
            tab_ref.at[pl.ds(rid, 1)], row_v.at[pl.ds(j, 1)], sem)

    for j in range(_BB):
        id_copy(uids_sm, uidtab_ref, urow_v, sem_u, j).start()
        id_copy(iids_sm, iidtab_ref, irow_v, sem_i, j).start()

    def onehot_emb(v_ref, tab_ref, n):
        v = v_ref[...]
        oh = (v == lax.broadcasted_iota(jnp.int32, (_BB, n), 1))
        return jnp.dot(oh.astype(_F32), tab_ref[...],
                       preferred_element_type=_F32)

    def side(x_ref, w_ref, b_ref, nrev):
        w = w_ref[...]
        bvec = b_ref[...]
        acc = jnp.zeros((_BB, _D), _F32)
        for r in range(nrev):
            x = x_ref[:, r, :]
            h = jnp.dot(x, w, preferred_element_type=_F32) + bvec
            acc = acc + jnp.maximum(h, 0.0)
        return acc

    uf_ref[:, 1, :] = onehot_emb(typ_ref, ttab_ref, _TYPE_N)
    if_ref[:, 1, :] = onehot_emb(mon_ref, mtab_ref, _MONTH_N)
    uf_ref[:, 2, :] = side(pu_ref, wu_ref, bu_ref, _UMR)
    if_ref[:, 2, :] = side(pi_ref, wi_ref, bi_ref, _IMR)

    for j in range(_BB):
        id_copy(uids_sm, uidtab_ref, urow_v, sem_u, j).wait()
        id_copy(iids_sm, iidtab_ref, irow_v, sem_i, j).wait()
    uf_ref[:, 0, :] = urow_v[...]
    if_ref[:, 0, :] = irow_v[...]


def _dense(pu3, pi3, Wu, Wi, bu2, bi2, uid_tab, iid_tab, t_tab, m_tab,
           uids, iids, typ2, mon2):
    grid = (_B // _BB,)
    grid_spec = pltpu.PrefetchScalarGridSpec(
        num_scalar_prefetch=2,
        grid=grid,
        in_specs=[
            pl.BlockSpec((_BB, _UMR, _D), lambda i, *_: (i, 0, 0)),
            pl.BlockSpec((_BB, _IMR, _D), lambda i, *_: (i, 0, 0)),
            pl.BlockSpec((_D, _D), lambda i, *_: (0, 0)),
            pl.BlockSpec((_D, _D), lambda i, *_: (0, 0)),
            pl.BlockSpec((1, _D), lambda i, *_: (0, 0)),
            pl.BlockSpec((1, _D), lambda i, *_: (0, 0)),
            pl.BlockSpec(memory_space=pl.ANY),
            pl.BlockSpec(memory_space=pl.ANY),
            pl.BlockSpec((_TYPE_N, _D), lambda i, *_: (0, 0)),
            pl.BlockSpec((_MONTH_N, _D), lambda i, *_: (0, 0)),
            pl.BlockSpec((_BB, 1), lambda i, *_: (i, 0)),
            pl.BlockSpec((_BB, 1), lambda i, *_: (i, 0)),
        ],
        out_specs=[
            pl.BlockSpec((_BB, 3, _D), lambda i, *_: (i, 0, 0)),
            pl.BlockSpec((_BB, 3, _D), lambda i, *_: (i, 0, 0)),
        ],
        scratch_shapes=[
            pltpu.VMEM((_BB, _D), _F32),
            pltpu.VMEM((_BB, _D), _F32),
            pltpu.SemaphoreType.DMA,
            pltpu.SemaphoreType.DMA,
        ],
    )
    return pl.pallas_call(
        _dense_body,
        grid_spec=grid_spec,
        out_shape=[
            jax.ShapeDtypeStruct((_B, 3, _D), _F32),
            jax.ShapeDtypeStruct((_B, 3, _D), _F32),
        ],
    )(uids, iids, pu3, pi3, Wu, Wi, bu2, bi2, uid_tab, iid_tab, t_tab, m_tab,
      typ2, mon2)


def kernel(user_reviews, item_reviews, uids, iids, user_item2id, item_user2id,
           user_item_ratio, item_user_ratio, user_doc, item_doc, type, month,
           user_word_table, item_word_table, user_id_table, item_id_table,
           type_table, month_table, Wu, bu, Wi, bi):
    ur_idx = user_reviews.reshape(-1).astype(jnp.int32)
    ir_idx = item_reviews.reshape(-1).astype(jnp.int32)
    pu, pi = _sc_pool(ur_idx, ir_idx, user_word_table, item_word_table)
    user_fea, item_fea = _dense(
        pu.reshape(_B, _UMR, _D), pi.reshape(_B, _IMR, _D),
        Wu, Wi, bu.reshape(1, _D), bi.reshape(1, _D),
        user_id_table, item_id_table, type_table, month_table,
        uids.astype(jnp.int32), iids.astype(jnp.int32),
        type.astype(jnp.int32).reshape(_B, 1),
        month.astype(jnp.int32).reshape(_B, 1))
    return (user_fea, item_fea)


# speedup vs baseline: 21.5687x; 1.1116x over previous
import functools

import jax
import jax.numpy as jnp
from jax import lax
from jax.experimental import pallas as pl
from jax.experimental.pallas import tpu as pltpu
from jax.experimental.pallas import tpu_sc as plsc

_B = 1024
_UMR = 20
_IMR = 20
_RML = 50
_D = 64
_TYPE_N = 10
_MONTH_N = 12
_NC = 2
_NS = 16
_NW = _NC * _NS
_BW = _B // _NW
_GB = 4
_ROWS = _GB * _RML
_BPB = _UMR // _GB
_NBLK = _BW * _BPB
_F32 = jnp.float32


def _build_sc_pool():
    mesh = plsc.VectorSubcoreMesh(core_axis_name="c", subcore_axis_name="s")
    out_type = (
        jax.ShapeDtypeStruct((_B, _UMR, _D), _F32),
        jax.ShapeDtypeStruct((_B, _IMR, _D), _F32),
    )
    scratch_types = [
        pltpu.VMEM((_BW, _UMR * _RML), jnp.int32),
        pltpu.VMEM((2, _ROWS, _D), _F32),
        pltpu.VMEM((_BW, _UMR, _D), _F32),
        pltpu.SemaphoreType.DMA,
        pltpu.SemaphoreType.DMA,
    ]

    @functools.partial(
        pl.kernel, out_type=out_type, mesh=mesh, scratch_types=scratch_types,
        compiler_params=pltpu.CompilerParams(use_tc_tiling_on_sc=False))
    def sc_pool(ur_idx, ir_idx, uwt, iwt,
                pu_out, pi_out,
                idx_v, rows_v, pool_v, sem_a, sem_b):
        wid = lax.axis_index("s") * _NC + lax.axis_index("c")
        sems = (sem_a, sem_b)

        def run_side(idx_hbm, tab_hbm, out_hbm):
            pltpu.sync_copy(idx_hbm.at[pl.ds(wid * _BW, _BW)], idx_v)

            def start(blk, b):
                bat = blk // _BPB
                o = (blk % _BPB) * _ROWS
                pltpu.make_async_copy(
                    tab_hbm.at[idx_v.at[bat].at[pl.ds(o, _ROWS)]],
                    rows_v.at[b], sems[b]).start()

            def wait(b):
                pltpu.make_async_copy(
                    tab_hbm.at[idx_v.at[0].at[pl.ds(0, _ROWS)]],
                    rows_v.at[b], sems[b]).wait()

            start(0, 0)
            start(1, 1)

            def block_body(t, carry):
                for b in range(2):
                    blk = 2 * t + b
                    wait(b)
                    rows = rows_v.at[b]
                    bat = blk // _BPB
                    r0 = (blk % _BPB) * _GB
                    for s in range(_GB):
                        base = s * _RML
                        acc = tuple(rows[base, pl.ds(c * 16, 16)]
                                    for c in range(4))

                        def row_body(t2, a, base=base, rows=rows):
                            r = 1 + 2 * t2
                            a = tuple(
                                a[c] + rows[base + r, pl.ds(c * 16, 16)]
                                for c in range(4))
                            return tuple(
                                a[c] + rows[base + r + 1, pl.ds(c * 16, 16)]
                                for c in range(4))

                        acc = lax.fori_loop(0, (_RML - 2) // 2, row_body, acc)
                        acc = tuple(
                            acc[c] + rows[base + _RML - 1, pl.ds(c * 16, 16)]
                            for c in range(4))
                        for c in range(4):
                            pool_v[bat, r0 + s, pl.ds(c * 16, 16)] = acc[c]
                    nblk = blk + 2

                    @pl.when(nblk < _NBLK)
                    def _():
                        start(nblk, b)
                return carry

            lax.fori_loop(0, _NBLK // 2, block_body, 0)
            pltpu.sync_copy(pool_v, out_hbm.at[pl.ds(wid * _BW, _BW)])

        run_side(ur_idx, uwt, pu_out)
        run_side(ir_idx, iwt, pi_out)

    return sc_pool


_sc_pool = _build_sc_pool()

_BB = 256


def _idgather_body(uids_sm, iids_sm, uidtab_ref, iidtab_ref,
                   ue_ref, ie_ref, sem_u, sem_i):
    i = pl.program_id(0)

    def id_copy(ids_sm, tab_ref, out_ref, sem, j):
        rid = ids_sm[i * _BB + j]
        return pltpu.make_async_copy(
            tab_ref.at[pl.ds(rid, 1)], out_ref.at[pl.ds(j, 1)], sem)

    for j in range(_BB):
        id_copy(uids_sm, uidtab_ref, ue_ref, sem_u, j).start()
        id_copy(iids_sm, iidtab_ref, ie_ref, sem_i, j).start()
    for j in range(_BB):
        id_copy(uids_sm, uidtab_ref, ue_ref, sem_u, j).wait()
        id_copy(iids_sm, iidtab_ref, ie_ref, sem_i, j).wait()


def _idgather(uid_tab, iid_tab, uids, iids):
    grid_spec = pltpu.PrefetchScalarGridSpec(
        num_scalar_prefetch=2,
        grid=(_B // _BB,),
        in_specs=[
            pl.BlockSpec(memory_space=pl.ANY),
            pl.BlockSpec(memory_space=pl.ANY),
        ],
        out_specs=[
            pl.BlockSpec((_BB, _D), lambda i, *_: (i, 0)),
            pl.BlockSpec((_BB, _D), lambda i, *_: (i, 0)),
        ],
        scratch_shapes=[
            pltpu.SemaphoreType.DMA,
            pltpu.SemaphoreType.DMA,
        ],
    )
    return pl.pallas_call(
        _idgather_body,
        grid_spec=grid_spec,
        out_shape=[
            jax.ShapeDtypeStruct((_B, _D), _F32),
            jax.ShapeDtypeStruct((_B, _D), _F32),
        ],
    )(uids, iids, uid_tab, iid_tab)


def _dense_body(pu_ref, pi_ref, wu_ref, wi_ref, bu_ref, bi_ref,
                ttab_ref, mtab_ref, typ_ref, mon_ref,
                ue_ref, ie_ref, uf_ref, if_ref):
    def onehot_emb(v_ref, tab_ref, n):
        v = v_ref[...]
        oh = (v == lax.broadcasted_iota(jnp.int32, (_BB, n), 1))
        return jnp.dot(oh.astype(_F32), tab_ref[...],
                       preferred_element_type=_F32)

    def side(x_ref, w_ref, b_ref, nrev):
        w = w_ref[...]
        bvec = b_ref[...]
        acc = jnp.zeros((_BB, _D), _F32)
        for r in range(nrev):
            x = x_ref[:, r, :]
            h = jnp.dot(x, w, preferred_element_type=_F32) + bvec
            acc = acc + jnp.maximum(h, 0.0)
        return acc

    uf_ref[:, 0, :] = ue_ref[...]
    if_ref[:, 0, :] = ie_ref[...]
    uf_ref[:, 1, :] = onehot_emb(typ_ref, ttab_ref, _TYPE_N)
    if_ref[:, 1, :] = onehot_emb(mon_ref, mtab_ref, _MONTH_N)
    uf_ref[:, 2, :] = side(pu_ref, wu_ref, bu_ref, _UMR)
    if_ref[:, 2, :] = side(pi_ref, wi_ref, bi_ref, _IMR)


def _dense(pu3, pi3, Wu, Wi, bu2, bi2, t_tab, m_tab, typ2, mon2, ue, ie):
    return pl.pallas_call(
        _dense_body,
        grid=(_B // _BB,),
        in_specs=[
            pl.BlockSpec((_BB, _UMR, _D), lambda i: (i, 0, 0)),
            pl.BlockSpec((_BB, _IMR, _D), lambda i: (i, 0, 0)),
            pl.BlockSpec((_D, _D), lambda i: (0, 0)),
            pl.BlockSpec((_D, _D), lambda i: (0, 0)),
            pl.BlockSpec((1, _D), lambda i: (0, 0)),
            pl.BlockSpec((1, _D), lambda i: (0, 0)),
            pl.BlockSpec((_TYPE_N, _D), lambda i: (0, 0)),
            pl.BlockSpec((_MONTH_N, _D), lambda i: (0, 0)),
            pl.BlockSpec((_BB, 1), lambda i: (i, 0)),
            pl.BlockSpec((_BB, 1), lambda i: (i, 0)),
            pl.BlockSpec((_BB, _D), lambda i: (i, 0)),
            pl.BlockSpec((_BB, _D), lambda i: (i, 0)),
        ],
        out_specs=[
            pl.BlockSpec((_BB, 3, _D), lambda i: (i, 0, 0)),
            pl.BlockSpec((_BB, 3, _D), lambda i: (i, 0, 0)),
        ],
        out_shape=[
            jax.ShapeDtypeStruct((_B, 3, _D), _F32),
            jax.ShapeDtypeStruct((_B, 3, _D), _F32),
        ],
    )(pu3, pi3, Wu, Wi, bu2, bi2, t_tab, m_tab, typ2, mon2, ue, ie)


def kernel(user_reviews, item_reviews, uids, iids, user_item2id, item_user2id,
           user_item_ratio, item_user_ratio, user_doc, item_doc, type, month,
           user_word_table, item_word_table, user_id_table, item_id_table,
           type_table, month_table, Wu, bu, Wi, bi):
    ue, ie = _idgather(user_id_table, item_id_table,
                       uids.astype(jnp.int32), iids.astype(jnp.int32))
    pu3, pi3 = _sc_pool(
        user_reviews.astype(jnp.int32).reshape(_B, _UMR * _RML),
        item_reviews.astype(jnp.int32).reshape(_B, _IMR * _RML),
        user_word_table, item_word_table)
    user_fea, item_fea = _dense(
        pu3, pi3, Wu, Wi, bu.reshape(1, _D), bi.reshape(1, _D),
        type_table, month_table,
        type.astype(jnp.int32).reshape(_B, 1),
        month.astype(jnp.int32).reshape(_B, 1), ue, ie)
    return (user_fea, item_fea)


# speedup vs baseline: 23.4407x; 1.0868x over previous
import functools

import jax
import jax.numpy as jnp
from jax import lax
from jax.experimental import pallas as pl
from jax.experimental.pallas import tpu as pltpu
from jax.experimental.pallas import tpu_sc as plsc

_B = 1024
_UMR = 20
_IMR = 20
_RML = 50
_D = 64
_TYPE_N = 10
_MONTH_N = 12
_NC = 2
_NS = 16
_NW = _NC * _NS
_BW = _B // _NW
_GB = 4
_ROWS = _GB * _RML
_BPB = _UMR // _GB
_NBLK = _BW * _BPB
_NBUF = 4
_F32 = jnp.float32


def _build_sc_pool():
    mesh = plsc.VectorSubcoreMesh(core_axis_name="c", subcore_axis_name="s")
    out_type = (
        jax.ShapeDtypeStruct((_B, _UMR, _D), _F32),
        jax.ShapeDtypeStruct((_B, _IMR, _D), _F32),
    )
    scratch_types = [
        pltpu.VMEM((_BW, _UMR, _RML), jnp.int32),
        pltpu.VMEM((_NBUF, _ROWS, _D), _F32),
        pltpu.VMEM((_BW, _UMR, _D), _F32),
        tuple(pltpu.SemaphoreType.DMA for _ in range(_NBUF)),
    ]

    @functools.partial(
        pl.kernel, out_type=out_type, mesh=mesh, scratch_types=scratch_types,
        compiler_params=pltpu.CompilerParams(use_tc_tiling_on_sc=False))
    def sc_pool(ur_idx, ir_idx, uwt, iwt,
                pu_out, pi_out,
                idx_v, rows_v, pool_v, sems):
        wid = lax.axis_index("s") * _NC + lax.axis_index("c")

        def run_side(idx_hbm, tab_hbm, out_hbm):
            pltpu.sync_copy(idx_hbm.at[pl.ds(wid * _BW, _BW)], idx_v)

            def start(blk, b):
                bat = blk // _BPB
                r0 = (blk % _BPB) * _GB
                for k in range(_GB):
                    pltpu.make_async_copy(
                        tab_hbm.at[idx_v.at[bat].at[r0 + k]],
                        rows_v.at[b].at[pl.ds(k * _RML, _RML)],
                        sems[b]).start()

            def wait(b):
                for k in range(_GB):
                    pltpu.make_async_copy(
                        tab_hbm.at[idx_v.at[0].at[0]],
                        rows_v.at[b].at[pl.ds(k * _RML, _RML)],
                        sems[b]).wait()

            for b in range(_NBUF):
                start(b, b)

            def block_body(t, carry):
                for b in range(_NBUF):
                    blk = _NBUF * t + b
                    wait(b)
                    rows = rows_v.at[b]
                    bat = blk // _BPB
                    r0 = (blk % _BPB) * _GB
                    for s in range(_GB):
                        base = s * _RML
                        acc = tuple(rows[base, pl.ds(c * 16, 16)]
                                    for c in range(4))

                        def row_body(t2, a, base=base, rows=rows):
                            r = 1 + 2 * t2
                            a = tuple(
                                a[c] + rows[base + r, pl.ds(c * 16, 16)]
                                for c in range(4))
                            return tuple(
                                a[c] + rows[base + r + 1, pl.ds(c * 16, 16)]
                                for c in range(4))

                        acc = lax.fori_loop(0, (_RML - 2) // 2, row_body, acc)
                        acc = tuple(
                            acc[c] + rows[base + _RML - 1, pl.ds(c * 16, 16)]
                            for c in range(4))
                        for c in range(4):
                            pool_v[bat, r0 + s, pl.ds(c * 16, 16)] = acc[c]
                    nblk = blk + _NBUF

                    @pl.when(nblk < _NBLK)
                    def _():
                        start(nblk, b)
                return carry

            lax.fori_loop(0, _NBLK // _NBUF, block_body, 0)
            pltpu.sync_copy(pool_v, out_hbm.at[pl.ds(wid * _BW, _BW)])

        run_side(ur_idx, uwt, pu_out)
        run_side(ir_idx, iwt, pi_out)

    return sc_pool


_sc_pool = _build_sc_pool()

_BB = 256


def _idgather_body(uids_sm, iids_sm, uidtab_ref, iidtab_ref,
                   ue_ref, ie_ref, sem_u, sem_i):
    i = pl.program_id(0)

    def id_copy(ids_sm, tab_ref, out_ref, sem, j):
        rid = ids_sm[i * _BB + j]
        return pltpu.make_async_copy(
            tab_ref.at[pl.ds(rid, 1)], out_ref.at[pl.ds(j, 1)], sem)

    for j in range(_BB):
        id_copy(uids_sm, uidtab_ref, ue_ref, sem_u, j).start()
        id_copy(iids_sm, iidtab_ref, ie_ref, sem_i, j).start()
    for j in range(_BB):
        id_copy(uids_sm, uidtab_ref, ue_ref, sem_u, j).wait()
        id_copy(iids_sm, iidtab_ref, ie_ref, sem_i, j).wait()


def _idgather(uid_tab, iid_tab, uids, iids):
    grid_spec = pltpu.PrefetchScalarGridSpec(
        num_scalar_prefetch=2,
        grid=(_B // _BB,),
        in_specs=[
            pl.BlockSpec(memory_space=pl.ANY),
            pl.BlockSpec(memory_space=pl.ANY),
        ],
        out_specs=[
            pl.BlockSpec((_BB, _D), lambda i, *_: (i, 0)),
            pl.BlockSpec((_BB, _D), lambda i, *_: (i, 0)),
        ],
        scratch_shapes=[
            pltpu.SemaphoreType.DMA,
            pltpu.SemaphoreType.DMA,
        ],
    )
    return pl.pallas_call(
        _idgather_body,
        grid_spec=grid_spec,
        out_shape=[
            jax.ShapeDtypeStruct((_B, _D), _F32),
            jax.ShapeDtypeStruct((_B, _D), _F32),
        ],
    )(uids, iids, uid_tab, iid_tab)


def _dense_body(pu_ref, pi_ref, wu_ref, wi_ref, bu_ref, bi_ref,
                ttab_ref, mtab_ref, typ_ref, mon_ref,
                ue_ref, ie_ref, uf_ref, if_ref):
    def onehot_emb(v_ref, tab_ref, n):
        v = v_ref[...]
        oh = (v == lax.broadcasted_iota(jnp.int32, (_BB, n), 1))
        return jnp.dot(oh.astype(_F32), tab_ref[...],
                       preferred_element_type=_F32)

    def side(x_ref, w_ref, b_ref, nrev):
        w = w_ref[...]
        bvec = b_ref[...]
        acc = jnp.zeros((_BB, _D), _F32)
        for r in range(nrev):
            x = x_ref[:, r, :]
            h = jnp.dot(x, w, preferred_element_type=_F32) + bvec
            acc = acc + jnp.maximum(h, 0.0)
        return acc

    uf_ref[:, 0, :] = ue_ref[...]
    if_ref[:, 0, :] = ie_ref[...]
    uf_ref[:, 1, :] = onehot_emb(typ_ref, ttab_ref, _TYPE_N)
    if_ref[:, 1, :] = onehot_emb(mon_ref, mtab_ref, _MONTH_N)
    uf_ref[:, 2, :] = side(pu_ref, wu_ref, bu_ref, _UMR)
    if_ref[:, 2, :] = side(pi_ref, wi_ref, bi_ref, _IMR)


def _dense(pu3, pi3, Wu, Wi, bu2, bi2, t_tab, m_tab, typ2, mon2, ue, ie):
    return pl.pallas_call(
        _dense_body,
        grid=(_B // _BB,),
        in_specs=[
            pl.BlockSpec((_BB, _UMR, _D), lambda i: (i, 0, 0)),
            pl.BlockSpec((_BB, _IMR, _D), lambda i: (i, 0, 0)),
            pl.BlockSpec((_D, _D), lambda i: (0, 0)),
            pl.BlockSpec((_D, _D), lambda i: (0, 0)),
            pl.BlockSpec((1, _D), lambda i: (0, 0)),
            pl.BlockSpec((1, _D), lambda i: (0, 0)),
            pl.BlockSpec((_TYPE_N, _D), lambda i: (0, 0)),
            pl.BlockSpec((_MONTH_N, _D), lambda i: (0, 0)),
            pl.BlockSpec((_BB, 1), lambda i: (i, 0)),
            pl.BlockSpec((_BB, 1), lambda i: (i, 0)),
            pl.BlockSpec((_BB, _D), lambda i: (i, 0)),
            pl.BlockSpec((_BB, _D), lambda i: (i, 0)),
        ],
        out_specs=[
            pl.BlockSpec((_BB, 3, _D), lambda i: (i, 0, 0)),
            pl.BlockSpec((_BB, 3, _D), lambda i: (i, 0, 0)),
        ],
        out_shape=[
            jax.ShapeDtypeStruct((_B, 3, _D), _F32),
            jax.ShapeDtypeStruct((_B, 3, _D), _F32),
        ],
    )(pu3, pi3, Wu, Wi, bu2, bi2, t_tab, m_tab, typ2, mon2, ue, ie)


def kernel(user_reviews, item_reviews, uids, iids, user_item2id, item_user2id,
           user_item_ratio, item_user_ratio, user_doc, item_doc, type, month,
           user_word_table, item_word_table, user_id_table, item_id_table,
           type_table, month_table, Wu, bu, Wi, bi):
    ue, ie = _idgather(user_id_table, item_id_table,
                       uids.astype(jnp.int32), iids.astype(jnp.int32))
    pu3, pi3 = _sc_pool(user_reviews.astype(jnp.int32),
                        item_reviews.astype(jnp.int32),
                        user_word_table, item_word_table)
    user_fea, item_fea = _dense(
        pu3, pi3, Wu, Wi, bu.reshape(1, _D), bi.reshape(1, _D),
        type_table, month_table,
        type.astype(jnp.int32).reshape(_B, 1),
        month.astype(jnp.int32).reshape(_B, 1), ue, ie)
    return (user_fea, item_fea)
